# baseline (speedup 1.0000x reference)
import functools

import jax
import jax.numpy as jnp
from jax import lax
from jax.experimental import pallas as pl
from jax.experimental.pallas import tpu as pltpu
from jax.experimental.pallas import tpu_sc as plsc

BATCH = 4096
HIST = 50
EMBED = 128
NUM_WORKERS = 32
NSPLIT = 4
CBATCH = BATCH // NSPLIT
ROWS_PER_W = CBATCH // NUM_WORKERS
K = 8
NGROUP = ROWS_PER_W // K
NBUF = 2

_mesh = plsc.VectorSubcoreMesh(core_axis_name="c", subcore_axis_name="s")


@functools.partial(
    pl.kernel,
    out_type=jax.ShapeDtypeStruct((CBATCH, HIST, EMBED), jnp.float32),
    mesh=_mesh,
    scratch_types=[
        pltpu.VMEM((ROWS_PER_W, HIST), jnp.int32),
        pltpu.VMEM((NBUF, K, HIST, EMBED), jnp.float32),
        pltpu.SemaphoreType.DMA,
        pltpu.SemaphoreType.DMA,
    ],
)
def _emb_gather(idx_hbm, table_hbm, out_hbm, idx_v, rows_v, gsem, ssem):
    wid = lax.axis_index("s") * 2 + lax.axis_index("c")
    base = wid * ROWS_PER_W
    pltpu.sync_copy(idx_hbm.at[pl.ds(base, ROWS_PER_W)], idx_v)

    def g_copy(g, k):
        return pltpu.make_async_copy(
            table_hbm.at[idx_v.at[g * K + k]],
            rows_v.at[g % NBUF, k], gsem)

    def s_copy(g):
        return pltpu.make_async_copy(
            rows_v.at[g % NBUF],
            out_hbm.at[pl.ds(base + g * K, K)], ssem)

    def start_group(g):
        for k in range(K):
            g_copy(g, k).start()

    def wait_group(g):
        for k in range(K):
            g_copy(g, k).wait()

    start_group(0)

    @pl.loop(0, NGROUP)
    def _body(g):
        @pl.when(g > 0)
        def _():
            s_copy(g - 1).wait()

        @pl.when(g + 1 < NGROUP)
        def _():
            start_group(g + 1)

        wait_group(g)
        s_copy(g).start()

    s_copy(NGROUP - 1).wait()


def kernel(x, table):
    x32 = x.astype(jnp.int32)
    parts = [_emb_gather(x32[i * CBATCH:(i + 1) * CBATCH], table)
             for i in range(NSPLIT)]
    return jnp.concatenate(parts, axis=0)


# speedup vs baseline: 3.2159x; 3.2159x over previous
import functools

import jax
import jax.numpy as jnp
from jax import lax
from jax.experimental import pallas as pl
from jax.experimental.pallas import tpu as pltpu
from jax.experimental.pallas import tpu_sc as plsc

BATCH = 4096
HIST = 50
EMBED = 128
NUM_WORKERS = 32
COLS_PER_W = BATCH // NUM_WORKERS
G = 2
NG = HIST // G
NBUF = 3

_mesh = plsc.VectorSubcoreMesh(core_axis_name="c", subcore_axis_name="s")


@functools.partial(
    pl.kernel,
    out_type=jax.ShapeDtypeStruct((HIST, BATCH, EMBED), jnp.float32),
    mesh=_mesh,
    scratch_types=[
        pltpu.VMEM((HIST, COLS_PER_W), jnp.int32),
        pltpu.VMEM((NBUF, G, COLS_PER_W, EMBED), jnp.float32),
        pltpu.SemaphoreType.DMA,
        pltpu.SemaphoreType.DMA,
    ],
)
def _emb_gather(idx_hbm, table_hbm, out_hbm, idx_v, rows_v, gsem, ssem):
    wid = lax.axis_index("s") * 2 + lax.axis_index("c")
    base = wid * COLS_PER_W
    pltpu.sync_copy(idx_hbm.at[:, pl.ds(base, COLS_PER_W)], idx_v)

    def g_copy(t, j):
        return pltpu.make_async_copy(
            table_hbm.at[idx_v.at[t * G + j]], rows_v.at[t % NBUF, j], gsem)

    def s_copy(t):
        return pltpu.make_async_copy(
            rows_v.at[t % NBUF],
            out_hbm.at[pl.ds(t * G, G), pl.ds(base, COLS_PER_W)], ssem)

    def start_group(t):
        for j in range(G):
            g_copy(t, j).start()

    def wait_group(t):
        for j in range(G):
            g_copy(t, j).wait()

    for t in range(NBUF - 1):
        start_group(t)

    @pl.loop(0, NG)
    def _body(t):
        @pl.when(t > 0)
        def _():
            s_copy(t - 1).wait()

        @pl.when(t + NBUF - 1 < NG)
        def _():
            start_group(t + NBUF - 1)

        wait_group(t)
        s_copy(t).start()

    s_copy(NG - 1).wait()


def kernel(x, table):
    xt = x.astype(jnp.int32).T
    out_t = _emb_gather(xt, table)
    return jnp.transpose(out_t, (1, 0, 2))
